# Initial kernel scaffold; baseline (speedup 1.0000x reference)
import jax
import jax.numpy as jnp
from jax.experimental import pallas as pl


def kernel(inputs, barycentric_coordinates, kernel):
    raise NotImplementedError("write your pallas kernel here")


# speedup vs baseline: 1.1044x; 1.1044x over previous
import functools

import jax
import jax.numpy as jnp
from jax import lax
from jax.experimental import pallas as pl
from jax.experimental.pallas import tpu as pltpu
from jax.experimental.pallas import tpu_sc as plsc

N_NODES = 10000
D = 128
N_RADIAL = 2
N_THETA = 4
V = N_RADIAL * N_THETA
OUT = 64

NC, NS = 2, 16
NW = NC * NS
NODES_PER_W = 320
N_PAD = NW * NODES_PER_W
CHUNK_NODES = 16
CHUNK_ROWS = CHUNK_NODES * V
N_CHUNKS = NODES_PER_W // CHUNK_NODES


def _sc_interp(table, idx, w):
    mesh = plsc.VectorSubcoreMesh(core_axis_name="c", subcore_axis_name="s")

    @functools.partial(
        pl.kernel,
        mesh=mesh,
        out_type=jax.ShapeDtypeStruct((N_PAD, V * D), jnp.float32),
        scratch_types=[
            pltpu.VMEM((3, N_CHUNKS, CHUNK_ROWS), jnp.int32),
            pltpu.VMEM((3, N_CHUNKS, CHUNK_ROWS), jnp.float32),
            pltpu.VMEM((CHUNK_ROWS, D), jnp.float32),
            pltpu.VMEM((CHUNK_ROWS, D), jnp.float32),
            pltpu.VMEM((CHUNK_ROWS, D), jnp.float32),
            pltpu.VMEM((CHUNK_NODES, V * D), jnp.float32),
            pltpu.SemaphoreType.DMA,
        ],
    )
    def k(table_hbm, idx_hbm, w_hbm, out_hbm, idx_v, w_v, g0, g1, g2, acc_v, sem):
        wid = lax.axis_index("s") * NC + lax.axis_index("c")
        pltpu.sync_copy(idx_hbm.at[wid], idx_v)
        pltpu.sync_copy(w_hbm.at[wid], w_v)

        def chunk_body(ci, carry):
            pltpu.async_copy(table_hbm.at[idx_v.at[0, ci]], g0, sem).wait()
            pltpu.async_copy(table_hbm.at[idx_v.at[1, ci]], g1, sem).wait()
            pltpu.async_copy(table_hbm.at[idx_v.at[2, ci]], g2, sem).wait()

            def group_body(g, c2):
                wv0 = w_v[0, ci, pl.ds(g * 16, 16)]
                wv1 = w_v[1, ci, pl.ds(g * 16, 16)]
                wv2 = w_v[2, ci, pl.ds(g * 16, 16)]
                for lane in range(16):
                    w0 = wv0[lane]
                    w1 = wv1[lane]
                    w2 = wv2[lane]
                    p = g * 16 + lane
                    nl = p // V
                    f0 = (lane % V) * D
                    for q in range(D // 16):
                        a = g0[p, pl.ds(q * 16, 16)]
                        b = g1[p, pl.ds(q * 16, 16)]
                        c = g2[p, pl.ds(q * 16, 16)]
                        acc_v[nl, pl.ds(f0 + q * 16, 16)] = (
                            a * w0 + b * w1 + c * w2)
                return c2

            lax.fori_loop(0, CHUNK_ROWS // 16, group_body, 0)
            n0 = wid * NODES_PER_W + ci * CHUNK_NODES
            pltpu.sync_copy(acc_v, out_hbm.at[pl.ds(n0, CHUNK_NODES)])
            return carry

        lax.fori_loop(0, N_CHUNKS, chunk_body, 0)

    return k(table, idx, w)


def _tc_conv(pull2d, wstack):
    BM = 400

    def body(x_ref, w_ref, o_ref):
        x = x_ref[...]
        acc = jnp.dot(x, w_ref[0], preferred_element_type=jnp.float32)
        for r in range(1, N_THETA):
            acc = jnp.maximum(
                acc, jnp.dot(x, w_ref[r], preferred_element_type=jnp.float32))
        o_ref[...] = jnp.maximum(acc, 0.0)

    return pl.pallas_call(
        body,
        grid=(N_NODES // BM,),
        in_specs=[
            pl.BlockSpec((BM, V * D), lambda i: (i, 0)),
            pl.BlockSpec((N_THETA, V * D, OUT), lambda i: (0, 0, 0)),
        ],
        out_specs=pl.BlockSpec((BM, OUT), lambda i: (i, 0)),
        out_shape=jax.ShapeDtypeStruct((N_NODES, OUT), jnp.float32),
    )(pull2d, wstack)


def kernel(inputs, barycentric_coordinates, kernel):
    bc = barycentric_coordinates
    w = bc[..., 2::2]
    idx = bc[..., 3::2].astype(jnp.int32)
    pad = N_PAD - N_NODES
    wp = jnp.pad(w, ((0, pad), (0, 0), (0, 0)))
    ip = jnp.pad(idx, ((0, pad), (0, 0), (0, 0)))
    wp = wp.reshape(NW, N_CHUNKS, CHUNK_ROWS, 3).transpose(0, 3, 1, 2)
    ip = ip.reshape(NW, N_CHUNKS, CHUNK_ROWS, 3).transpose(0, 3, 1, 2)
    pull = _sc_interp(inputs, ip, wp)

    kf = kernel.reshape(V, OUT, D)
    rots = []
    for r in range(N_THETA):
        sl = [(vv // N_THETA) * N_THETA + ((vv % N_THETA) + r) % N_THETA
              for vv in range(V)]
        rots.append(kf[jnp.array(sl)].transpose(0, 2, 1).reshape(V * D, OUT))
    wstack = jnp.stack(rots, axis=0)
    return _tc_conv(pull, wstack)


# speedup vs baseline: 1.4057x; 1.2728x over previous
import functools

import jax
import jax.numpy as jnp
from jax import lax
from jax.experimental import pallas as pl
from jax.experimental.pallas import tpu as pltpu
from jax.experimental.pallas import tpu_sc as plsc

N_NODES = 10000
D = 128
N_RADIAL = 2
N_THETA = 4
V = N_RADIAL * N_THETA
OUT = 64

NC, NS = 2, 16
NW = NC * NS
NODES_PER_W = 320
N_PAD = NW * NODES_PER_W
CHUNK_NODES = 8
CHUNK_ROWS = CHUNK_NODES * V
N_CHUNKS = NODES_PER_W // CHUNK_NODES
NGROUPS = CHUNK_ROWS // 16


def _sc_interp(table, idx, w):
    mesh = plsc.VectorSubcoreMesh(core_axis_name="c", subcore_axis_name="s")

    @functools.partial(
        pl.kernel,
        mesh=mesh,
        out_type=jax.ShapeDtypeStruct((N_PAD, V * D), jnp.float32),
        scratch_types=[
            pltpu.VMEM((3, N_CHUNKS, CHUNK_ROWS), jnp.int32),
            pltpu.VMEM((3, N_CHUNKS, CHUNK_ROWS), jnp.float32),
            pltpu.VMEM((CHUNK_ROWS, D), jnp.float32),
            pltpu.VMEM((CHUNK_ROWS, D), jnp.float32),
            pltpu.VMEM((CHUNK_ROWS, D), jnp.float32),
            pltpu.VMEM((CHUNK_ROWS, D), jnp.float32),
            pltpu.VMEM((CHUNK_ROWS, D), jnp.float32),
            pltpu.VMEM((CHUNK_ROWS, D), jnp.float32),
            pltpu.VMEM((CHUNK_NODES, V * D), jnp.float32),
            pltpu.VMEM((CHUNK_NODES, V * D), jnp.float32),
            pltpu.SemaphoreType.DMA,
            pltpu.SemaphoreType.DMA,
            pltpu.SemaphoreType.DMA,
            pltpu.SemaphoreType.DMA,
        ],
    )
    def k(table_hbm, idx_hbm, w_hbm, out_hbm, idx_v, w_v,
          ga0, ga1, ga2, gb0, gb1, gb2, acca, accb,
          semga, semgb, semoa, semob):
        wid = lax.axis_index("s") * NC + lax.axis_index("c")
        pltpu.sync_copy(idx_hbm.at[wid], idx_v)
        pltpu.sync_copy(w_hbm.at[wid], w_v)

        def fire_gathers(ci, g0, g1, g2, sem):
            pltpu.async_copy(table_hbm.at[idx_v.at[0, ci]], g0, sem)
            pltpu.async_copy(table_hbm.at[idx_v.at[1, ci]], g1, sem)
            pltpu.async_copy(table_hbm.at[idx_v.at[2, ci]], g2, sem)

        def wait_gathers(g0, g1, g2, sem):
            src = table_hbm.at[pl.ds(0, CHUNK_ROWS)]
            pltpu.make_async_copy(src, g0, sem).wait()
            pltpu.make_async_copy(src, g1, sem).wait()
            pltpu.make_async_copy(src, g2, sem).wait()

        def drain_out(acc, sem):
            pltpu.make_async_copy(
                acc, out_hbm.at[pl.ds(0, CHUNK_NODES)], sem).wait()

        def compute_chunk(ci, g0, g1, g2, acc):
            @plsc.parallel_loop(0, NGROUPS)
            def group_body(g):
                wv0 = w_v[0, ci, pl.ds(g * 16, 16)]
                wv1 = w_v[1, ci, pl.ds(g * 16, 16)]
                wv2 = w_v[2, ci, pl.ds(g * 16, 16)]
                for lane in range(16):
                    w0 = wv0[lane]
                    w1 = wv1[lane]
                    w2 = wv2[lane]
                    p = g * 16 + lane
                    nl = 2 * g + lane // V
                    f0 = (lane % V) * D
                    avs = [g0[p, pl.ds(q * 16, 16)] for q in range(D // 16)]
                    bvs = [g1[p, pl.ds(q * 16, 16)] for q in range(D // 16)]
                    cvs = [g2[p, pl.ds(q * 16, 16)] for q in range(D // 16)]
                    for q in range(D // 16):
                        acc[nl, pl.ds(f0 + q * 16, 16)] = (
                            avs[q] * w0 + bvs[q] * w1 + cvs[q] * w2)

        def do_chunk(ci, c2, g0, g1, g2, acc, semg, semo):
            wait_gathers(g0, g1, g2, semg)

            @pl.when(c2 >= 1)
            def _():
                drain_out(acc, semo)

            compute_chunk(ci, g0, g1, g2, acc)
            n0 = wid * NODES_PER_W + ci * CHUNK_NODES
            pltpu.async_copy(acc, out_hbm.at[pl.ds(n0, CHUNK_NODES)], semo)

            @pl.when(ci + 2 < N_CHUNKS)
            def _():
                fire_gathers(ci + 2, g0, g1, g2, semg)

        fire_gathers(0, ga0, ga1, ga2, semga)
        fire_gathers(1, gb0, gb1, gb2, semgb)

        def pair_body(c2, carry):
            do_chunk(2 * c2, c2, ga0, ga1, ga2, acca, semga, semoa)
            do_chunk(2 * c2 + 1, c2, gb0, gb1, gb2, accb, semgb, semob)
            return carry

        lax.fori_loop(0, N_CHUNKS // 2, pair_body, 0)
        drain_out(acca, semoa)
        drain_out(accb, semob)

    return k(table, idx, w)


def _tc_conv(pull2d, wstack):
    BM = 400

    def body(x_ref, w_ref, o_ref):
        x = x_ref[...]
        acc = jnp.dot(x, w_ref[0], preferred_element_type=jnp.float32)
        for r in range(1, N_THETA):
            acc = jnp.maximum(
                acc, jnp.dot(x, w_ref[r], preferred_element_type=jnp.float32))
        o_ref[...] = jnp.maximum(acc, 0.0)

    return pl.pallas_call(
        body,
        grid=(N_NODES // BM,),
        in_specs=[
            pl.BlockSpec((BM, V * D), lambda i: (i, 0)),
            pl.BlockSpec((N_THETA, V * D, OUT), lambda i: (0, 0, 0)),
        ],
        out_specs=pl.BlockSpec((BM, OUT), lambda i: (i, 0)),
        out_shape=jax.ShapeDtypeStruct((N_NODES, OUT), jnp.float32),
    )(pull2d, wstack)


def kernel(inputs, barycentric_coordinates, kernel):
    bc = barycentric_coordinates
    w = bc[..., 2::2]
    idx = bc[..., 3::2].astype(jnp.int32)
    pad = N_PAD - N_NODES
    wp = jnp.pad(w, ((0, pad), (0, 0), (0, 0)))
    ip = jnp.pad(idx, ((0, pad), (0, 0), (0, 0)))
    wp = wp.reshape(NW, N_CHUNKS, CHUNK_ROWS, 3).transpose(0, 3, 1, 2)
    ip = ip.reshape(NW, N_CHUNKS, CHUNK_ROWS, 3).transpose(0, 3, 1, 2)
    pull = _sc_interp(inputs, ip, wp)

    kf = kernel.reshape(V, OUT, D)
    rots = []
    for r in range(N_THETA):
        sl = [(vv // N_THETA) * N_THETA + ((vv % N_THETA) + r) % N_THETA
              for vv in range(V)]
        rots.append(kf[jnp.array(sl)].transpose(0, 2, 1).reshape(V * D, OUT))
    wstack = jnp.stack(rots, axis=0)
    return _tc_conv(pull, wstack)
